# Initial kernel scaffold; baseline (speedup 1.0000x reference)
import jax
import jax.numpy as jnp
from jax.experimental import pallas as pl


def kernel(x, edge_index, W1, b1, W2, b2):
    raise NotImplementedError("write your pallas kernel here")


# speedup vs baseline: 7.0409x; 7.0409x over previous
import functools

import jax
import jax.numpy as jnp
from jax import lax
from jax.experimental import pallas as pl
from jax.experimental.pallas import tpu as pltpu
from jax.experimental.pallas import tpu_sc as plsc

N = 10000
E = 160000
F = 256
H = 128
NC = 2
NS = 16
L = 16

CH = 128
EP = 163840
EROWS = EP // CH
SROWS = EROWS // NS
DROWS = EROWS // (NC * NS)
NPAD = N + 48
SLICE = 640
LAST = N - 15 * SLICE
ZR = 80

_sc_mesh = plsc.VectorSubcoreMesh(
    core_axis_name="c", subcore_axis_name="s", num_cores=NC, num_subcores=NS
)


def _each_slice(s, fn):
    @pl.when(s < NS - 1)
    def _():
        for t in range(SLICE // ZR):
            fn(s * SLICE + t * ZR, ZR)

    @pl.when(s == NS - 1)
    def _():
        for t in range(LAST // ZR):
            fn((NS - 1) * SLICE + t * ZR, ZR)


def _deg_body(dst_hbm, out_hbm, dstb, ones, zb, acc):
    c = lax.axis_index("c")
    s = lax.axis_index("s")
    wid = c * NS + s

    def fill_ones(i, carry):
        for k in range(H // L):
            ones[i, pl.ds(k * L, L)] = jnp.ones((L,), jnp.float32)
        return carry

    lax.fori_loop(0, CH, fill_ones, 0)

    def fill_z(i, carry):
        for k in range(H // L):
            zb[i, pl.ds(k * L, L)] = jnp.zeros((L,), jnp.float32)
        return carry

    lax.fori_loop(0, ZR, fill_z, 0)

    _each_slice(s, lambda r0, nr: pltpu.sync_copy(zb, acc.at[pl.ds(r0, nr)]))
    plsc.subcore_barrier()

    pltpu.sync_copy(dst_hbm.at[pl.ds(wid * DROWS, DROWS)], dstb)

    def chunk(j, carry):
        pltpu.sync_copy(ones, acc.at[dstb.at[j]], add=True)
        return carry

    lax.fori_loop(0, DROWS, chunk, 0)
    plsc.subcore_barrier()

    _each_slice(
        s,
        lambda r0, nr: pltpu.sync_copy(
            acc.at[pl.ds(r0, nr)], out_hbm.at[pl.ds(c * N + r0, nr)]
        ),
    )


def _scatter_body(src_hbm, dst_hbm, g_hbm, out_hbm, srcb, dstb, gidx, rows, zb, acc, sem):
    c = lax.axis_index("c")
    s = lax.axis_index("s")

    def fill_z(i, carry):
        for k in range(H // L):
            zb[i, pl.ds(k * L, L)] = jnp.zeros((L,), jnp.float32)
        return carry

    lax.fori_loop(0, ZR, fill_z, 0)
    _each_slice(s, lambda r0, nr: pltpu.sync_copy(zb, acc.at[pl.ds(r0, nr)]))
    plsc.subcore_barrier()

    pltpu.sync_copy(src_hbm.at[pl.ds(s * SROWS, SROWS)], srcb)
    pltpu.sync_copy(dst_hbm.at[pl.ds(s * SROWS, SROWS)], dstb)
    base = c * N

    def chunk(j, carry):
        for k in range(CH // L):
            gidx[pl.ds(k * L, L)] = srcb[j, pl.ds(k * L, L)] + base
        pltpu.async_copy(g_hbm.at[gidx], rows, sem).wait()
        pltpu.sync_copy(rows, acc.at[dstb.at[j]], add=True)
        return carry

    lax.fori_loop(0, SROWS, chunk, 0)
    plsc.subcore_barrier()

    _each_slice(
        s,
        lambda r0, nr: pltpu.sync_copy(
            acc.at[pl.ds(r0, nr)], out_hbm.at[pl.ds(c * N + r0, nr)]
        ),
    )


def _make_deg_kernel(interpret=False):
    return functools.partial(
        pl.kernel,
        out_type=jax.ShapeDtypeStruct((NC * N, H), jnp.float32),
        mesh=_sc_mesh,
        scratch_types=[
            pltpu.VMEM((DROWS, CH), jnp.int32),
            pltpu.VMEM((CH, H), jnp.float32),
            pltpu.VMEM((ZR, H), jnp.float32),
            pltpu.VMEM_SHARED((NPAD, H), jnp.float32),
        ],
        interpret=interpret,
    )(_deg_body)


def _make_scatter_kernel(interpret=False):
    return functools.partial(
        pl.kernel,
        out_type=jax.ShapeDtypeStruct((NC * N, H), jnp.float32),
        mesh=_sc_mesh,
        scratch_types=[
            pltpu.VMEM((SROWS, CH), jnp.int32),
            pltpu.VMEM((SROWS, CH), jnp.int32),
            pltpu.VMEM((CH,), jnp.int32),
            pltpu.VMEM((CH, H), jnp.float32),
            pltpu.VMEM((ZR, H), jnp.float32),
            pltpu.VMEM_SHARED((NPAD, H), jnp.float32),
            pltpu.SemaphoreType.DMA,
        ],
        interpret=interpret,
    )(_scatter_body)


_deg_kernel = _make_deg_kernel()
_scatter_kernel = _make_scatter_kernel()


BM = 1000
NRB = N // BM


def _dinv(parts_ref):
    deg = parts_ref[0, :, 0:1] + parts_ref[1, :, 0:1] + 1.0
    return lax.rsqrt(deg)


def _mm_body(parts_ref, x_ref, w_ref, out_ref):
    out_ref[...] = _dinv(parts_ref) * jnp.dot(
        x_ref[...], w_ref[...], preferred_element_type=jnp.float32
    )


def _mm_call(parts2, x, w):
    return pl.pallas_call(
        _mm_body,
        grid=(NC, NRB),
        in_specs=[
            pl.BlockSpec((NC, BM, H), lambda h, i: (0, i, 0)),
            pl.BlockSpec((BM, F), lambda h, i: (i, 0)),
            pl.BlockSpec((F, H), lambda h, i: (0, h)),
        ],
        out_specs=pl.BlockSpec((BM, H), lambda h, i: (h * NRB + i, 0)),
        out_shape=jax.ShapeDtypeStruct((NC * N, H), jnp.float32),
    )(parts2, x, w)


def _comb1_body(s_ref, g_ref, parts_ref, b_ref, out_ref):
    t = s_ref[...] + g_ref[...]
    xc = jnp.concatenate([t[0], t[1]], axis=1)
    out_ref[...] = jnp.maximum(_dinv(parts_ref) * xc + b_ref[...], 0.0)


def _comb1_call(s2d, g2d, parts2, b):
    return pl.pallas_call(
        _comb1_body,
        grid=(NRB,),
        in_specs=[
            pl.BlockSpec((NC, BM, H), lambda i: (0, i, 0)),
            pl.BlockSpec((NC, BM, H), lambda i: (0, i, 0)),
            pl.BlockSpec((NC, BM, H), lambda i: (0, i, 0)),
            pl.BlockSpec((1, F), lambda i: (0, 0)),
        ],
        out_specs=pl.BlockSpec((BM, F), lambda i: (i, 0)),
        out_shape=jax.ShapeDtypeStruct((N, F), jnp.float32),
    )(s2d, g2d, parts2, b)


def _comb2_body(s_ref, g_ref, parts_ref, b_ref, x1_ref, out_ref):
    t = s_ref[...] + g_ref[...]
    xc = jnp.concatenate([t[0], t[1]], axis=1)
    x2 = jnp.maximum(_dinv(parts_ref) * xc + b_ref[...], 0.0)
    out_ref[...] = jnp.maximum(x1_ref[...], x2)


def _comb2_call(s2d, g2d, parts2, b, x1):
    return pl.pallas_call(
        _comb2_body,
        grid=(NRB,),
        in_specs=[
            pl.BlockSpec((NC, BM, H), lambda i: (0, i, 0)),
            pl.BlockSpec((NC, BM, H), lambda i: (0, i, 0)),
            pl.BlockSpec((NC, BM, H), lambda i: (0, i, 0)),
            pl.BlockSpec((1, F), lambda i: (0, 0)),
            pl.BlockSpec((BM, F), lambda i: (i, 0)),
        ],
        out_specs=pl.BlockSpec((BM, F), lambda i: (i, 0)),
        out_shape=jax.ShapeDtypeStruct((N, F), jnp.float32),
    )(s2d, g2d, parts2, b, x1)


def kernel(x, edge_index, W1, b1, W2, b2):
    src = edge_index[0].astype(jnp.int32)
    dst = edge_index[1].astype(jnp.int32)
    pad = EP - E
    src_p = jnp.concatenate([src, jnp.zeros((pad,), jnp.int32)])
    dst_p = jnp.concatenate([dst, jnp.full((pad,), N, jnp.int32)])
    src2d = src_p.reshape(EROWS, CH)
    dst2d = dst_p.reshape(EROWS, CH)
    b1r = b1.reshape(1, F)
    b2r = b2.reshape(1, F)

    parts = _deg_kernel(dst2d)
    parts2 = parts.reshape(NC, N, H)

    g1 = _mm_call(parts2, x, W1)
    s1 = _scatter_kernel(src2d, dst2d, g1)
    x1 = _comb1_call(
        s1.reshape(NC, N, H), g1.reshape(NC, N, H), parts2, b1r
    )

    g2 = _mm_call(parts2, x1, W2)
    s2 = _scatter_kernel(src2d, dst2d, g2)
    out = _comb2_call(
        s2.reshape(NC, N, H), g2.reshape(NC, N, H), parts2, b2r, x1
    )
    return out


# speedup vs baseline: 8.4954x; 1.2066x over previous
import functools

import jax
import jax.numpy as jnp
from jax import lax
from jax.experimental import pallas as pl
from jax.experimental.pallas import tpu as pltpu
from jax.experimental.pallas import tpu_sc as plsc

N = 10000
E = 160000
F = 256
H = 128
NC = 2
NS = 16
L = 16

CH = 128
EP = 163840
EROWS = EP // CH
SROWS = EROWS // NS
DROWS = EROWS // (NC * NS)
NPAD = N + 48
SLICE = 640
LAST = N - 15 * SLICE
ZR = 80
HROWS = 40

_sc_mesh = plsc.VectorSubcoreMesh(
    core_axis_name="c", subcore_axis_name="s", num_cores=NC, num_subcores=NS
)


def _each_slice(s, fn):
    @pl.when(s < NS - 1)
    def _():
        for t in range(SLICE // ZR):
            fn(s * SLICE + t * ZR, ZR)

    @pl.when(s == NS - 1)
    def _():
        for t in range(LAST // ZR):
            fn((NS - 1) * SLICE + t * ZR, ZR)


def _deg_body(dst_hbm, out_hbm, dstb, ones, zb, acc):
    c = lax.axis_index("c")
    s = lax.axis_index("s")
    wid = c * NS + s

    def fill_ones(i, carry):
        for k in range(H // L):
            ones[i, pl.ds(k * L, L)] = jnp.ones((L,), jnp.float32)
        return carry

    lax.fori_loop(0, CH, fill_ones, 0)

    def fill_z(i, carry):
        for k in range(H // L):
            zb[i, pl.ds(k * L, L)] = jnp.zeros((L,), jnp.float32)
        return carry

    lax.fori_loop(0, ZR, fill_z, 0)

    _each_slice(s, lambda r0, nr: pltpu.sync_copy(zb, acc.at[pl.ds(r0, nr)]))
    plsc.subcore_barrier()

    pltpu.sync_copy(dst_hbm.at[pl.ds(wid * DROWS, DROWS)], dstb)

    def chunk(j, carry):
        pltpu.sync_copy(ones, acc.at[dstb.at[j]], add=True)
        return carry

    lax.fori_loop(0, DROWS, chunk, 0)
    plsc.subcore_barrier()

    _each_slice(
        s,
        lambda r0, nr: pltpu.sync_copy(
            acc.at[pl.ds(r0, nr)], out_hbm.at[pl.ds(c * N + r0, nr)]
        ),
    )


def _scatter_body(
    src_hbm, dst_hbm, g_hbm, out_hbm,
    srcb, dstb, gidx0, gidx1, rows0, rows1, acc, sem0, sem1,
):
    c = lax.axis_index("c")
    s = lax.axis_index("s")

    def fill_z(i, carry):
        for k in range(H // L):
            rows0[i, pl.ds(k * L, L)] = jnp.zeros((L,), jnp.float32)
        return carry

    lax.fori_loop(0, ZR, fill_z, 0)
    _each_slice(
        s, lambda r0, nr: pltpu.sync_copy(rows0.at[pl.ds(0, nr)], acc.at[pl.ds(r0, nr)])
    )
    plsc.subcore_barrier()
    base = c * N

    def start(j, gidx, rows, sem):
        for k in range(CH // L):
            gidx[pl.ds(k * L, L)] = srcb[j, pl.ds(k * L, L)] + base
        return pltpu.async_copy(g_hbm.at[gidx], rows, sem)

    def finish(j, rows, copy):
        copy.wait()
        pltpu.sync_copy(rows, acc.at[dstb.at[j]], add=True)

    for half in range(2):
        row0 = s * SROWS + half * HROWS
        pltpu.sync_copy(src_hbm.at[pl.ds(row0, HROWS)], srcb)
        pltpu.sync_copy(dst_hbm.at[pl.ds(row0, HROWS)], dstb)

        cp0 = start(0, gidx0, rows0, sem0)

        @pl.loop(0, HROWS - 2, step=2)
        def _(j0):
            cp1 = start(j0 + 1, gidx1, rows1, sem1)
            finish(j0, rows0, cp0)
            start(j0 + 2, gidx0, rows0, sem0)
            finish(j0 + 1, rows1, cp1)

        cp1 = start(HROWS - 1, gidx1, rows1, sem1)
        finish(HROWS - 2, rows0, cp0)
        finish(HROWS - 1, rows1, cp1)
    plsc.subcore_barrier()

    _each_slice(
        s,
        lambda r0, nr: pltpu.sync_copy(
            acc.at[pl.ds(r0, nr)], out_hbm.at[pl.ds(c * N + r0, nr)]
        ),
    )


def _make_deg_kernel(interpret=False):
    return functools.partial(
        pl.kernel,
        out_type=jax.ShapeDtypeStruct((NC * N, H), jnp.float32),
        mesh=_sc_mesh,
        scratch_types=[
            pltpu.VMEM((DROWS, CH), jnp.int32),
            pltpu.VMEM((CH, H), jnp.float32),
            pltpu.VMEM((ZR, H), jnp.float32),
            pltpu.VMEM_SHARED((NPAD, H), jnp.float32),
        ],
        interpret=interpret,
    )(_deg_body)


def _make_scatter_kernel(interpret=False):
    return functools.partial(
        pl.kernel,
        out_type=jax.ShapeDtypeStruct((NC * N, H), jnp.float32),
        mesh=_sc_mesh,
        scratch_types=[
            pltpu.VMEM((HROWS, CH), jnp.int32),
            pltpu.VMEM((HROWS, CH), jnp.int32),
            pltpu.VMEM((CH,), jnp.int32),
            pltpu.VMEM((CH,), jnp.int32),
            pltpu.VMEM((CH, H), jnp.float32),
            pltpu.VMEM((CH, H), jnp.float32),
            pltpu.VMEM_SHARED((NPAD, H), jnp.float32),
            pltpu.SemaphoreType.DMA,
            pltpu.SemaphoreType.DMA,
        ],
        interpret=interpret,
    )(_scatter_body)


_deg_kernel = _make_deg_kernel()
_scatter_kernel = _make_scatter_kernel()


BM = 1000
NRB = N // BM


def _dinv(parts_ref):
    deg = parts_ref[0, :, 0:1] + parts_ref[1, :, 0:1] + 1.0
    return lax.rsqrt(deg)


def _mm_body(parts_ref, x_ref, w_ref, out_ref):
    out_ref[...] = _dinv(parts_ref) * jnp.dot(
        x_ref[...], w_ref[...], preferred_element_type=jnp.float32
    )


def _mm_call(parts2, x, w):
    return pl.pallas_call(
        _mm_body,
        grid=(NC, NRB),
        in_specs=[
            pl.BlockSpec((NC, BM, H), lambda h, i: (0, i, 0)),
            pl.BlockSpec((BM, F), lambda h, i: (i, 0)),
            pl.BlockSpec((F, H), lambda h, i: (0, h)),
        ],
        out_specs=pl.BlockSpec((BM, H), lambda h, i: (h * NRB + i, 0)),
        out_shape=jax.ShapeDtypeStruct((NC * N, H), jnp.float32),
    )(parts2, x, w)


def _comb1_body(s_ref, g_ref, parts_ref, b_ref, out_ref):
    t = s_ref[...] + g_ref[...]
    xc = jnp.concatenate([t[0], t[1]], axis=1)
    out_ref[...] = jnp.maximum(_dinv(parts_ref) * xc + b_ref[...], 0.0)


def _comb1_call(s2d, g2d, parts2, b):
    return pl.pallas_call(
        _comb1_body,
        grid=(NRB,),
        in_specs=[
            pl.BlockSpec((NC, BM, H), lambda i: (0, i, 0)),
            pl.BlockSpec((NC, BM, H), lambda i: (0, i, 0)),
            pl.BlockSpec((NC, BM, H), lambda i: (0, i, 0)),
            pl.BlockSpec((1, F), lambda i: (0, 0)),
        ],
        out_specs=pl.BlockSpec((BM, F), lambda i: (i, 0)),
        out_shape=jax.ShapeDtypeStruct((N, F), jnp.float32),
    )(s2d, g2d, parts2, b)


def _comb2_body(s_ref, g_ref, parts_ref, b_ref, x1_ref, out_ref):
    t = s_ref[...] + g_ref[...]
    xc = jnp.concatenate([t[0], t[1]], axis=1)
    x2 = jnp.maximum(_dinv(parts_ref) * xc + b_ref[...], 0.0)
    out_ref[...] = jnp.maximum(x1_ref[...], x2)


def _comb2_call(s2d, g2d, parts2, b, x1):
    return pl.pallas_call(
        _comb2_body,
        grid=(NRB,),
        in_specs=[
            pl.BlockSpec((NC, BM, H), lambda i: (0, i, 0)),
            pl.BlockSpec((NC, BM, H), lambda i: (0, i, 0)),
            pl.BlockSpec((NC, BM, H), lambda i: (0, i, 0)),
            pl.BlockSpec((1, F), lambda i: (0, 0)),
            pl.BlockSpec((BM, F), lambda i: (i, 0)),
        ],
        out_specs=pl.BlockSpec((BM, F), lambda i: (i, 0)),
        out_shape=jax.ShapeDtypeStruct((N, F), jnp.float32),
    )(s2d, g2d, parts2, b, x1)


def kernel(x, edge_index, W1, b1, W2, b2):
    src = edge_index[0].astype(jnp.int32)
    dst = edge_index[1].astype(jnp.int32)
    pad = EP - E
    src_p = jnp.concatenate([src, jnp.zeros((pad,), jnp.int32)])
    dst_p = jnp.concatenate([dst, jnp.full((pad,), N, jnp.int32)])
    src2d = src_p.reshape(EROWS, CH)
    dst2d = dst_p.reshape(EROWS, CH)
    b1r = b1.reshape(1, F)
    b2r = b2.reshape(1, F)

    parts = _deg_kernel(dst2d)
    parts2 = parts.reshape(NC, N, H)

    g1 = _mm_call(parts2, x, W1)
    s1 = _scatter_kernel(src2d, dst2d, g1)
    x1 = _comb1_call(
        s1.reshape(NC, N, H), g1.reshape(NC, N, H), parts2, b1r
    )

    g2 = _mm_call(parts2, x1, W2)
    s2 = _scatter_kernel(src2d, dst2d, g2)
    out = _comb2_call(
        s2.reshape(NC, N, H), g2.reshape(NC, N, H), parts2, b2r, x1
    )
    return out


# speedup vs baseline: 8.6951x; 1.0235x over previous
import functools

import jax
import jax.numpy as jnp
from jax import lax
from jax.experimental import pallas as pl
from jax.experimental.pallas import tpu as pltpu
from jax.experimental.pallas import tpu_sc as plsc

N = 10000
E = 160000
F = 256
H = 128
NC = 2
NS = 16
L = 16

CH = 64
EP = 163840
EROWS = EP // CH
SROWS = EROWS // NS
DROWS = EROWS // (NC * NS)
NPAD = N + 8
SLICE = 640
LAST = N - 15 * SLICE
ZR = 80
PKS = 14
PKM = 1 << PKS
PKB = SROWS // 2

_sc_mesh = plsc.VectorSubcoreMesh(
    core_axis_name="c", subcore_axis_name="s", num_cores=NC, num_subcores=NS
)


def _each_slice(s, fn):
    @pl.when(s < NS - 1)
    def _():
        for t in range(SLICE // ZR):
            fn(s * SLICE + t * ZR, ZR)

    @pl.when(s == NS - 1)
    def _():
        for t in range(LAST // ZR):
            fn((NS - 1) * SLICE + t * ZR, ZR)


def _deg_body(pk_hbm, out_hbm, pkb, didx, ones, zb, acc):
    c = lax.axis_index("c")
    s = lax.axis_index("s")
    wid = c * NS + s

    def fill_ones(i, carry):
        for k in range(H // L):
            ones[i, pl.ds(k * L, L)] = jnp.ones((L,), jnp.float32)
        return carry

    lax.fori_loop(0, CH, fill_ones, 0)

    def fill_z(i, carry):
        for k in range(H // L):
            zb[i, pl.ds(k * L, L)] = jnp.zeros((L,), jnp.float32)
        return carry

    lax.fori_loop(0, ZR, fill_z, 0)

    _each_slice(s, lambda r0, nr: pltpu.sync_copy(zb, acc.at[pl.ds(r0, nr)]))
    plsc.subcore_barrier()

    pltpu.sync_copy(pk_hbm.at[pl.ds(wid * DROWS, DROWS)], pkb)

    def chunk(j, carry):
        for k in range(CH // L):
            didx[pl.ds(k * L, L)] = lax.shift_right_logical(
                pkb[j, pl.ds(k * L, L)], PKS
            )
        pltpu.sync_copy(ones, acc.at[didx], add=True)
        return carry

    lax.fori_loop(0, DROWS, chunk, 0)
    plsc.subcore_barrier()

    _each_slice(
        s,
        lambda r0, nr: pltpu.sync_copy(
            acc.at[pl.ds(r0, nr)], out_hbm.at[pl.ds(c * N + r0, nr)]
        ),
    )


def _scatter_body(
    pk_hbm, g_hbm, out_hbm,
    pkb, gidx, didx, rows, acc, gs0, gs1, gs2, gs3, ss0, ss1, ss2, ss3,
):
    c = lax.axis_index("c")
    s = lax.axis_index("s")
    gsem = (gs0, gs1, gs2, gs3)
    ssem = (ss0, ss1, ss2, ss3)

    def fill_z(i, carry):
        for k in range(H // L):
            rows[0, i, pl.ds(k * L, L)] = jnp.zeros((L,), jnp.float32)
        return carry

    lax.fori_loop(0, ZR, fill_z, 0)
    _each_slice(
        s,
        lambda r0, nr: pltpu.sync_copy(rows.at[0, pl.ds(0, nr)], acc.at[pl.ds(r0, nr)]),
    )
    plsc.subcore_barrier()
    base = c * N

    pltpu.sync_copy(pk_hbm.at[pl.ds(s * SROWS, PKB)], pkb)

    def start_gather(j, b):
        jr = jnp.where(j >= PKB, j - PKB, j)
        for k in range(CH // L):
            v = pkb[jr, pl.ds(k * L, L)]
            gidx[b, pl.ds(k * L, L)] = (v & (PKM - 1)) + base
            didx[b, pl.ds(k * L, L)] = lax.shift_right_logical(v, PKS)
        return pltpu.async_copy(g_hbm.at[gidx.at[b]], rows.at[b], gsem[b])

    def start_scatter(b):
        return pltpu.async_copy(rows.at[b], acc.at[didx.at[b]], ssem[b], add=True)

    def wait_gather(b):
        pltpu.make_async_copy(g_hbm.at[gidx.at[b]], rows.at[b], gsem[b]).wait()

    def wait_scatter(b):
        pltpu.make_async_copy(rows.at[b], acc.at[didx.at[b]], ssem[b]).wait()

    start_gather(0, 0)
    start_gather(1, 1)
    wait_gather(0)
    start_scatter(0)
    start_gather(2, 2)
    wait_gather(1)
    start_scatter(1)
    start_gather(3, 3)

    @pl.loop(2, SROWS - 2, step=4)
    def _(j0):
        @pl.when(j0 == PKB - 2)
        def _():
            pltpu.sync_copy(pk_hbm.at[pl.ds(s * SROWS + PKB, PKB)], pkb)

        for u in range(4):
            b = (2 + u) % 4
            wait_gather(b)
            start_scatter(b)
            bn = (b + 2) % 4
            wait_scatter(bn)
            start_gather(j0 + u + 2, bn)

    wait_gather(2)
    start_scatter(2)
    wait_gather(3)
    start_scatter(3)
    for b in range(4):
        wait_scatter(b)
    plsc.subcore_barrier()

    _each_slice(
        s,
        lambda r0, nr: pltpu.sync_copy(
            acc.at[pl.ds(r0, nr)], out_hbm.at[pl.ds(c * N + r0, nr)]
        ),
    )


def _make_deg_kernel(interpret=False):
    return functools.partial(
        pl.kernel,
        out_type=jax.ShapeDtypeStruct((NC * N, H), jnp.float32),
        mesh=_sc_mesh,
        scratch_types=[
            pltpu.VMEM((DROWS, CH), jnp.int32),
            pltpu.VMEM((CH,), jnp.int32),
            pltpu.VMEM((CH, H), jnp.float32),
            pltpu.VMEM((ZR, H), jnp.float32),
            pltpu.VMEM_SHARED((NPAD, H), jnp.float32),
        ],
        interpret=interpret,
    )(_deg_body)


def _make_scatter_kernel(interpret=False):
    return functools.partial(
        pl.kernel,
        out_type=jax.ShapeDtypeStruct((NC * N, H), jnp.float32),
        mesh=_sc_mesh,
        scratch_types=[
            pltpu.VMEM((PKB, CH), jnp.int32),
            pltpu.VMEM((4, CH), jnp.int32),
            pltpu.VMEM((4, CH), jnp.int32),
            pltpu.VMEM((4, CH, H), jnp.float32),
            pltpu.VMEM_SHARED((NPAD, H), jnp.float32),
            pltpu.SemaphoreType.DMA,
            pltpu.SemaphoreType.DMA,
            pltpu.SemaphoreType.DMA,
            pltpu.SemaphoreType.DMA,
            pltpu.SemaphoreType.DMA,
            pltpu.SemaphoreType.DMA,
            pltpu.SemaphoreType.DMA,
            pltpu.SemaphoreType.DMA,
        ],
        interpret=interpret,
    )(_scatter_body)


_deg_kernel = _make_deg_kernel()
_scatter_kernel = _make_scatter_kernel()


BM = 1000
NRB = N // BM


def _dinv(parts_ref):
    deg = parts_ref[0, :, 0:1] + parts_ref[1, :, 0:1] + 1.0
    return lax.rsqrt(deg)


def _mm_body(parts_ref, x_ref, w_ref, out_ref):
    out_ref[...] = _dinv(parts_ref) * jnp.dot(
        x_ref[...], w_ref[...], preferred_element_type=jnp.float32
    )


def _mm_call(parts2, x, w):
    return pl.pallas_call(
        _mm_body,
        grid=(NC, NRB),
        in_specs=[
            pl.BlockSpec((NC, BM, H), lambda h, i: (0, i, 0)),
            pl.BlockSpec((BM, F), lambda h, i: (i, 0)),
            pl.BlockSpec((F, H), lambda h, i: (0, h)),
        ],
        out_specs=pl.BlockSpec((BM, H), lambda h, i: (h * NRB + i, 0)),
        out_shape=jax.ShapeDtypeStruct((NC * N, H), jnp.float32),
    )(parts2, x, w)


def _comb1_body(s_ref, g_ref, parts_ref, b_ref, out_ref):
    t = s_ref[...] + g_ref[...]
    xc = jnp.concatenate([t[0], t[1]], axis=1)
    out_ref[...] = jnp.maximum(_dinv(parts_ref) * xc + b_ref[...], 0.0)


def _comb1_call(s2d, g2d, parts2, b):
    return pl.pallas_call(
        _comb1_body,
        grid=(NRB,),
        in_specs=[
            pl.BlockSpec((NC, BM, H), lambda i: (0, i, 0)),
            pl.BlockSpec((NC, BM, H), lambda i: (0, i, 0)),
            pl.BlockSpec((NC, BM, H), lambda i: (0, i, 0)),
            pl.BlockSpec((1, F), lambda i: (0, 0)),
        ],
        out_specs=pl.BlockSpec((BM, F), lambda i: (i, 0)),
        out_shape=jax.ShapeDtypeStruct((N, F), jnp.float32),
    )(s2d, g2d, parts2, b)


def _comb2_body(s_ref, g_ref, parts_ref, b_ref, x1_ref, out_ref):
    t = s_ref[...] + g_ref[...]
    xc = jnp.concatenate([t[0], t[1]], axis=1)
    x2 = jnp.maximum(_dinv(parts_ref) * xc + b_ref[...], 0.0)
    out_ref[...] = jnp.maximum(x1_ref[...], x2)


def _comb2_call(s2d, g2d, parts2, b, x1):
    return pl.pallas_call(
        _comb2_body,
        grid=(NRB,),
        in_specs=[
            pl.BlockSpec((NC, BM, H), lambda i: (0, i, 0)),
            pl.BlockSpec((NC, BM, H), lambda i: (0, i, 0)),
            pl.BlockSpec((NC, BM, H), lambda i: (0, i, 0)),
            pl.BlockSpec((1, F), lambda i: (0, 0)),
            pl.BlockSpec((BM, F), lambda i: (i, 0)),
        ],
        out_specs=pl.BlockSpec((BM, F), lambda i: (i, 0)),
        out_shape=jax.ShapeDtypeStruct((N, F), jnp.float32),
    )(s2d, g2d, parts2, b, x1)


def kernel(x, edge_index, W1, b1, W2, b2):
    src = edge_index[0].astype(jnp.int32)
    dst = edge_index[1].astype(jnp.int32)
    pad = EP - E
    src_p = jnp.concatenate([src, jnp.zeros((pad,), jnp.int32)])
    dst_p = jnp.concatenate([dst, jnp.full((pad,), N, jnp.int32)])
    pk2d = (dst_p * PKM + src_p).reshape(EROWS, CH)
    b1r = b1.reshape(1, F)
    b2r = b2.reshape(1, F)

    parts = _deg_kernel(pk2d)
    parts2 = parts.reshape(NC, N, H)

    g1 = _mm_call(parts2, x, W1)
    s1 = _scatter_kernel(pk2d, g1)
    x1 = _comb1_call(
        s1.reshape(NC, N, H), g1.reshape(NC, N, H), parts2, b1r
    )

    g2 = _mm_call(parts2, x1, W2)
    s2 = _scatter_kernel(pk2d, g2)
    out = _comb2_call(
        s2.reshape(NC, N, H), g2.reshape(NC, N, H), parts2, b2r, x1
    )
    return out
